# Initial kernel scaffold; baseline (speedup 1.0000x reference)
import jax
import jax.numpy as jnp
from jax.experimental import pallas as pl


def kernel(x, weight):
    raise NotImplementedError("write your pallas kernel here")


# speedup vs baseline: 1.2603x; 1.2603x over previous
import functools

import jax
import jax.numpy as jnp
from jax import lax
from jax.experimental import pallas as pl
from jax.experimental.pallas import tpu as pltpu
from jax.experimental.pallas import tpu_sc as plsc

_DIM = 4096
_NE = 64
_TOPK = 8
_T = 16384
_BT = 512

_NC = 2
_NS = 16
_NW = _NC * _NS
_R = _T // _NW


def _matmul_body(x_ref, w_ref, o_ref):
    o_ref[...] = lax.dot_general(
        x_ref[...], w_ref[...],
        dimension_numbers=(((1,), (1,)), ((), ())),
        preferred_element_type=jnp.float32,
    )


def _scores_tc(x, weight):
    return pl.pallas_call(
        _matmul_body,
        grid=(_T // _BT,),
        in_specs=[
            pl.BlockSpec((_BT, _DIM), lambda i: (i, 0)),
            pl.BlockSpec((_NE, _DIM), lambda i: (0, 0)),
        ],
        out_specs=pl.BlockSpec((_BT, _NE), lambda i: (i, 0)),
        out_shape=jax.ShapeDtypeStruct((_T, _NE), jnp.float32),
    )(x, weight)


def _topk_body(scores_hbm, wout_hbm, iout_hbm, sbuf, wbuf, ibuf):
    wid = lax.axis_index("s") * _NC + lax.axis_index("c")
    base = wid * _R
    pltpu.sync_copy(scores_hbm.at[pl.ds(base, _R)], sbuf)

    lanes = lax.iota(jnp.int32, 16)
    in_lo = lanes < 8

    def _merge(ka, va, kb, vb):
        kb_r = lax.rev(kb, (0,))
        vb_r = lax.rev(vb, (0,))
        k = jnp.where(in_lo, ka, kb_r)
        v = jnp.where(in_lo, va, vb_r)
        return plsc.sort_key_val(k, v, descending=True)

    def _row(r, carry):
        srt = []
        for j in range(4):
            k = sbuf[r, pl.ds(16 * j, 16)]
            srt.append(plsc.sort_key_val(k, lanes + 16 * j, descending=True))
        k01, v01 = _merge(*srt[0], *srt[1])
        k23, v23 = _merge(*srt[2], *srt[3])
        kf, vf = _merge(k01, v01, k23, v23)
        m = jnp.max(kf)
        e = jnp.exp(kf - m)
        e = jnp.where(in_lo, e, 0.0)
        s = jnp.broadcast_to(jnp.sum(e), (16,))
        wbuf[r] = e / s
        ibuf[r] = vf
        return carry

    lax.fori_loop(0, _R, _row, 0)
    pltpu.sync_copy(wbuf, wout_hbm.at[pl.ds(base, _R)])
    pltpu.sync_copy(ibuf, iout_hbm.at[pl.ds(base, _R)])


_topk_sc = functools.partial(
    pl.kernel,
    out_type=(
        jax.ShapeDtypeStruct((_T, 16), jnp.float32),
        jax.ShapeDtypeStruct((_T, 16), jnp.int32),
    ),
    mesh=plsc.VectorSubcoreMesh(core_axis_name="c", subcore_axis_name="s"),
    compiler_params=pltpu.CompilerParams(
        needs_layout_passes=False, use_tc_tiling_on_sc=False),
    scratch_types=[
        pltpu.VMEM((_R, _NE), jnp.float32),
        pltpu.VMEM((_R, 16), jnp.float32),
        pltpu.VMEM((_R, 16), jnp.int32),
    ],
)(_topk_body)


def kernel(x, weight):
    scores = _scores_tc(x, weight)
    w16, i16 = _topk_sc(scores)
    return (w16[:, :_TOPK], i16[:, :_TOPK])


# speedup vs baseline: 1.5072x; 1.1959x over previous
import functools

import jax
import jax.numpy as jnp
from jax import lax
from jax.experimental import pallas as pl
from jax.experimental.pallas import tpu as pltpu
from jax.experimental.pallas import tpu_sc as plsc

_DIM = 4096
_NE = 64
_TOPK = 8
_T = 16384
_BT = 512

_NC = 2
_NS = 16
_NW = _NC * _NS
_R = _T // _NW


def _matmul_body(x_ref, w_ref, o_ref):
    o_ref[...] = lax.dot_general(
        x_ref[...], w_ref[...],
        dimension_numbers=(((1,), (1,)), ((), ())),
        preferred_element_type=jnp.float32,
    )


def _scores_tc(x, weight):
    return pl.pallas_call(
        _matmul_body,
        grid=(_T // _BT,),
        in_specs=[
            pl.BlockSpec((_BT, _DIM), lambda i: (i, 0)),
            pl.BlockSpec((_NE, _DIM), lambda i: (0, 0)),
        ],
        out_specs=pl.BlockSpec((_BT, _NE), lambda i: (i, 0)),
        out_shape=jax.ShapeDtypeStruct((_T, _NE), jnp.float32),
    )(x, weight)


def _topk_body(scores_hbm, wout_hbm, iout_hbm, sbuf, wbuf, ibuf):
    wid = lax.axis_index("s") * _NC + lax.axis_index("c")
    base = wid * _R
    pltpu.sync_copy(scores_hbm.at[pl.ds(base, _R)], sbuf)

    lanes = lax.iota(jnp.int32, 16)
    in_lo = lanes < 8

    def _merge(ka, va, kb, vb):
        kb_r = lax.rev(kb, (0,))
        vb_r = lax.rev(vb, (0,))
        k = jnp.where(in_lo, ka, kb_r)
        v = jnp.where(in_lo, va, vb_r)
        return plsc.sort_key_val(k, v, descending=True)

    @plsc.parallel_loop(0, _R, step=1, unroll=8)
    def _row(r):
        srt = []
        for j in range(4):
            k = sbuf[r, pl.ds(16 * j, 16)]
            srt.append(plsc.sort_key_val(k, lanes + 16 * j, descending=True))
        k01, v01 = _merge(*srt[0], *srt[1])
        k23, v23 = _merge(*srt[2], *srt[3])
        kf, vf = _merge(k01, v01, k23, v23)
        m = jnp.max(kf)
        e = jnp.exp(kf - m)
        e = jnp.where(in_lo, e, 0.0)
        s = jnp.broadcast_to(jnp.sum(e), (16,))
        wbuf[r] = e / s
        ibuf[r] = vf
    pltpu.sync_copy(wbuf, wout_hbm.at[pl.ds(base, _R)])
    pltpu.sync_copy(ibuf, iout_hbm.at[pl.ds(base, _R)])


_topk_sc = functools.partial(
    pl.kernel,
    out_type=(
        jax.ShapeDtypeStruct((_T, 16), jnp.float32),
        jax.ShapeDtypeStruct((_T, 16), jnp.int32),
    ),
    mesh=plsc.VectorSubcoreMesh(core_axis_name="c", subcore_axis_name="s"),
    compiler_params=pltpu.CompilerParams(
        needs_layout_passes=False, use_tc_tiling_on_sc=False),
    scratch_types=[
        pltpu.VMEM((_R, _NE), jnp.float32),
        pltpu.VMEM((_R, 16), jnp.float32),
        pltpu.VMEM((_R, 16), jnp.int32),
    ],
)(_topk_body)


def kernel(x, weight):
    scores = _scores_tc(x, weight)
    w16, i16 = _topk_sc(scores)
    return (w16[:, :_TOPK], i16[:, :_TOPK])
